# baseline (speedup 1.0000x reference)
import functools

import jax
import jax.numpy as jnp
from jax.experimental import pallas as pl

_DN = (((1,), (0,)), ((), ()))
_IMAX = 2147483647


def _split2(x):
    x1 = x.astype(jnp.bfloat16)
    x2 = (x - x1.astype(jnp.float32)).astype(jnp.bfloat16)
    return x1, x2


def _ref_sum64(sq):
    w = sq.reshape(8, 8, sq.shape[-1])
    a = w[:, 0:4, :] + w[:, 4:8, :]
    a = a[:, 0:2, :] + a[:, 2:4, :]
    a = a[:, 0:1, :] + a[:, 1:2, :]
    s = a[0]
    for g in range(1, 8):
        s = s + a[g]
    return s


def _vq_body(z, la_cat, ct):
    t = z.shape[-1]
    nch = ct.shape[0]
    ones = jnp.ones((1, t), jnp.float32)
    rhs = jnp.concatenate([z, ones], axis=0)
    r1, r2 = _split2(rhs)
    rcat = jnp.concatenate([r1, r2, r1], axis=0)
    d = jax.lax.dot_general(la_cat, rcat, _DN,
                            preferred_element_type=jnp.float32)
    iota = jax.lax.broadcasted_iota(jnp.int32, d.shape, 0)
    bits = jax.lax.bitcast_convert_type(d, jnp.int32)
    kk = jax.lax.bitcast_convert_type((bits & -512) | iota, jnp.float32)

    def top(kc):
        kmin = jnp.min(kc, axis=0, keepdims=True)
        i = jax.lax.bitcast_convert_type(kmin, jnp.int32) & 511
        return i, jnp.where(kc == kmin, jnp.inf, kc)

    i1, kk = top(kk)
    i2, kk = top(kk)
    i3, _ = top(kk)

    def gather(i):
        lob = jnp.broadcast_to(i & 127, (nch, t))
        hib = jnp.broadcast_to(i >> 7, (nch, t))
        p0 = jnp.take_along_axis(ct[:, 0:128], lob, axis=1)
        p1 = jnp.take_along_axis(ct[:, 128:256], lob, axis=1)
        p2 = jnp.take_along_axis(ct[:, 256:384], lob, axis=1)
        p3 = jnp.take_along_axis(ct[:, 384:512], lob, axis=1)
        g01 = jnp.where(hib == 0, p0, p1)
        g23 = jnp.where(hib == 2, p2, p3)
        return jnp.where(hib <= 1, g01, g23)

    g1, g2, g3 = gather(i1), gather(i2), gather(i3)
    e1, e2, e3 = z - g1, z - g2, z - g3
    d1 = _ref_sum64(e1 * e1)
    d2 = _ref_sum64(e2 * e2)
    d3 = _ref_sum64(e3 * e3)

    dcat = jnp.concatenate([d1, d2, d3], axis=0)
    icat = jnp.concatenate([i1, i2, i3], axis=0)
    dmin = jnp.min(dcat, axis=0, keepdims=True)
    idx = jnp.min(jnp.where(dcat == dmin, icat, 512), axis=0, keepdims=True)
    e1b = jnp.broadcast_to(idx == i1, (nch, t))
    e2b = jnp.broadcast_to(idx == i2, (nch, t))
    g = jnp.where(e1b, g1, jnp.where(e2b, g2, g3))
    return z + (g - z), idx


def _vq_tc_kernel(z_ref, cb_ref, zq_ref, idx_ref):
    cb = cb_ref[...]
    cn = jnp.sum(cb * cb, axis=1, keepdims=True)
    laug = jnp.concatenate([cb * (-2.0), cn], axis=1)
    la1, la2 = _split2(laug)
    la_cat = jnp.concatenate([la1, la1, la2], axis=1)
    ct = cb.T
    for j in range(z_ref.shape[0]):
        zq, idx = _vq_body(z_ref[j], la_cat, ct)
        zq_ref[j] = zq
        idx_ref[j] = idx


@functools.partial(jax.jit, static_argnames=("interpret",))
def kernel(z_e, codebook, interpret=False):
    b, c, h, w = z_e.shape
    hw = h * w
    nc = codebook.shape[0]
    bb = 4
    z3 = z_e.reshape(b, c, hw)
    zq3, idx3 = pl.pallas_call(
        _vq_tc_kernel,
        grid=(b // bb,),
        in_specs=[
            pl.BlockSpec((bb, c, hw), lambda i: (i, 0, 0)),
            pl.BlockSpec((nc, c), lambda i: (0, 0)),
        ],
        out_specs=[
            pl.BlockSpec((bb, c, hw), lambda i: (i, 0, 0)),
            pl.BlockSpec((bb, 1, hw), lambda i: (i, 0, 0)),
        ],
        out_shape=[
            jax.ShapeDtypeStruct((b, c, hw), jnp.float32),
            jax.ShapeDtypeStruct((b, 1, hw), jnp.int32),
        ],
        interpret=interpret,
    )(z3, codebook)
    return zq3.reshape(b, c, h, w), idx3.reshape(b, h, w)


# speedup vs baseline: 1.0112x; 1.0112x over previous
import functools

import jax
import jax.numpy as jnp
from jax.experimental import pallas as pl

_DN = (((1,), (0,)), ((), ()))
_IMAX = 2147483647


def _split2(x):
    x1 = x.astype(jnp.bfloat16)
    x2 = (x - x1.astype(jnp.float32)).astype(jnp.bfloat16)
    return x1, x2


def _ref_sum64(sq):
    w = sq.reshape(8, 8, sq.shape[-1])
    a = w[:, 0:4, :] + w[:, 4:8, :]
    a = a[:, 0:2, :] + a[:, 2:4, :]
    a = a[:, 0:1, :] + a[:, 1:2, :]
    s = a[0]
    for g in range(1, 8):
        s = s + a[g]
    return s


def _vq_body(z, la_cat, ct):
    t = z.shape[-1]
    nch = z.shape[0]
    ones = jnp.ones((1, t), jnp.float32)
    rhs = jnp.concatenate([z, ones], axis=0)
    r1, r2 = _split2(rhs)
    rcat = jnp.concatenate([r1, r2, r1], axis=0)
    d = jax.lax.dot_general(la_cat, rcat, _DN,
                            preferred_element_type=jnp.float32)
    iota = jax.lax.broadcasted_iota(jnp.int32, d.shape, 0)
    bits = jax.lax.bitcast_convert_type(d, jnp.int32)
    kk = jax.lax.bitcast_convert_type((bits & -512) | iota, jnp.float32)

    def top(kc):
        kmin = jnp.min(kc, axis=0, keepdims=True)
        i = jax.lax.bitcast_convert_type(kmin, jnp.int32) & 511
        return i, jnp.where(kc == kmin, jnp.inf, kc)

    i1, kk = top(kk)
    i2, kk = top(kk)
    i3, _ = top(kk)

    ch, cm, cl = ct
    iota512 = jax.lax.broadcasted_iota(jnp.int32, (ch.shape[1], t), 0)

    def gather(i):
        oh = jnp.where(iota512 == i, 1.0, 0.0).astype(jnp.bfloat16)
        p = jax.lax.dot_general(ch, oh, _DN, preferred_element_type=jnp.float32)
        q = jax.lax.dot_general(cm, oh, _DN, preferred_element_type=jnp.float32)
        r = jax.lax.dot_general(cl, oh, _DN, preferred_element_type=jnp.float32)
        return (p + q) + r

    g1, g2, g3 = gather(i1), gather(i2), gather(i3)
    e1, e2, e3 = z - g1, z - g2, z - g3
    d1 = _ref_sum64(e1 * e1)
    d2 = _ref_sum64(e2 * e2)
    d3 = _ref_sum64(e3 * e3)

    dcat = jnp.concatenate([d1, d2, d3], axis=0)
    icat = jnp.concatenate([i1, i2, i3], axis=0)
    dmin = jnp.min(dcat, axis=0, keepdims=True)
    idx = jnp.min(jnp.where(dcat == dmin, icat, 512), axis=0, keepdims=True)
    e1b = jnp.broadcast_to(idx == i1, (nch, t))
    e2b = jnp.broadcast_to(idx == i2, (nch, t))
    g = jnp.where(e1b, g1, jnp.where(e2b, g2, g3))
    return z + (g - z), idx


def _vq_tc_kernel(z_ref, cb_ref, zq_ref, idx_ref):
    cb = cb_ref[...]
    cn = jnp.sum(cb * cb, axis=1, keepdims=True)
    laug = jnp.concatenate([cb * (-2.0), cn], axis=1)
    la1, la2 = _split2(laug)
    la_cat = jnp.concatenate([la1, la1, la2], axis=1)
    cbt = cb.T
    c1 = cbt.astype(jnp.bfloat16)
    r = cbt - c1.astype(jnp.float32)
    c2 = r.astype(jnp.bfloat16)
    c3 = (r - c2.astype(jnp.float32)).astype(jnp.bfloat16)
    ct = (c1, c2, c3)
    for j in range(z_ref.shape[0]):
        zq, idx = _vq_body(z_ref[j], la_cat, ct)
        zq_ref[j] = zq
        idx_ref[j] = idx


@functools.partial(jax.jit, static_argnames=("interpret",))
def kernel(z_e, codebook, interpret=False):
    b, c, h, w = z_e.shape
    hw = h * w
    nc = codebook.shape[0]
    bb = 4
    z3 = z_e.reshape(b, c, hw)
    zq3, idx3 = pl.pallas_call(
        _vq_tc_kernel,
        grid=(b // bb,),
        in_specs=[
            pl.BlockSpec((bb, c, hw), lambda i: (i, 0, 0)),
            pl.BlockSpec((nc, c), lambda i: (0, 0)),
        ],
        out_specs=[
            pl.BlockSpec((bb, c, hw), lambda i: (i, 0, 0)),
            pl.BlockSpec((bb, 1, hw), lambda i: (i, 0, 0)),
        ],
        out_shape=[
            jax.ShapeDtypeStruct((b, c, hw), jnp.float32),
            jax.ShapeDtypeStruct((b, 1, hw), jnp.int32),
        ],
        interpret=interpret,
    )(z3, codebook)
    return zq3.reshape(b, c, h, w), idx3.reshape(b, h, w)


# speedup vs baseline: 1.0149x; 1.0037x over previous
import functools

import jax
import jax.numpy as jnp
from jax.experimental import pallas as pl

_DN = (((1,), (0,)), ((), ()))
_IMAX = 2147483647


def _split2(x):
    x1 = x.astype(jnp.bfloat16)
    x2 = (x - x1.astype(jnp.float32)).astype(jnp.bfloat16)
    return x1, x2


def _ref_sum64(sq):
    w = sq.reshape(8, 8, sq.shape[-1])
    a = w[:, 0:4, :] + w[:, 4:8, :]
    a = a[:, 0:2, :] + a[:, 2:4, :]
    a = a[:, 0:1, :] + a[:, 1:2, :]
    s = a[0]
    for g in range(1, 8):
        s = s + a[g]
    return s


def _vq_body(z, la_cat, ct):
    t = z.shape[-1]
    nch = z.shape[0]
    ones = jnp.ones((1, t), jnp.float32)
    rhs = jnp.concatenate([z, ones], axis=0)
    r1, r2 = _split2(rhs)
    rcat = jnp.concatenate([r1, r2, r1], axis=0)
    d = jax.lax.dot_general(la_cat, rcat, _DN,
                            preferred_element_type=jnp.float32)
    iota = jax.lax.broadcasted_iota(jnp.int32, d.shape, 0)
    bits = jax.lax.bitcast_convert_type(d, jnp.int32)
    kk = jax.lax.bitcast_convert_type((bits & -512) | iota, jnp.float32)

    def top(kc):
        kmin = jnp.min(kc, axis=0, keepdims=True)
        i = jax.lax.bitcast_convert_type(kmin, jnp.int32) & 511
        return i, jnp.where(kc == kmin, jnp.inf, kc)

    i1, kk = top(kk)
    i2, kk = top(kk)
    i3, _ = top(kk)

    ch, cm, cl = ct
    iota512 = jax.lax.broadcasted_iota(jnp.int32, (ch.shape[1], t), 0)

    def gather(i):
        oh = jnp.where(iota512 == i, 1.0, 0.0).astype(jnp.bfloat16)
        p = jax.lax.dot_general(ch, oh, _DN, preferred_element_type=jnp.float32)
        q = jax.lax.dot_general(cm, oh, _DN, preferred_element_type=jnp.float32)
        r = jax.lax.dot_general(cl, oh, _DN, preferred_element_type=jnp.float32)
        return (p + q) + r

    g1, g2, g3 = gather(i1), gather(i2), gather(i3)
    e1, e2, e3 = z - g1, z - g2, z - g3
    d1 = _ref_sum64(e1 * e1)
    d2 = _ref_sum64(e2 * e2)
    d3 = _ref_sum64(e3 * e3)

    dcat = jnp.concatenate([d1, d2, d3], axis=0)
    icat = jnp.concatenate([i1, i2, i3], axis=0)
    dmin = jnp.min(dcat, axis=0, keepdims=True)
    idx = jnp.min(jnp.where(dcat == dmin, icat, 512), axis=0, keepdims=True)
    e1b = jnp.broadcast_to(idx == i1, (nch, t))
    e2b = jnp.broadcast_to(idx == i2, (nch, t))
    g = jnp.where(e1b, g1, jnp.where(e2b, g2, g3))
    return z + (g - z), idx


def _vq_tc_kernel(z_ref, cb_ref, zq_ref, idx_ref):
    cb = cb_ref[...]
    cn = jnp.sum(cb * cb, axis=1, keepdims=True)
    laug = jnp.concatenate([cb * (-2.0), cn], axis=1)
    la1, la2 = _split2(laug)
    la_cat = jnp.concatenate([la1, la1, la2], axis=1)
    cbt = cb.T
    c1 = cbt.astype(jnp.bfloat16)
    r = cbt - c1.astype(jnp.float32)
    c2 = r.astype(jnp.bfloat16)
    c3 = (r - c2.astype(jnp.float32)).astype(jnp.bfloat16)
    ct = (c1, c2, c3)
    for j in range(z_ref.shape[0]):
        zq, idx = _vq_body(z_ref[j], la_cat, ct)
        zq_ref[j] = zq
        idx_ref[j] = idx


@functools.partial(jax.jit, static_argnames=("interpret",))
def kernel(z_e, codebook, interpret=False):
    b, c, h, w = z_e.shape
    hw = h * w
    nc = codebook.shape[0]
    bb = 8
    z3 = z_e.reshape(b, c, hw)
    zq3, idx3 = pl.pallas_call(
        _vq_tc_kernel,
        grid=(b // bb,),
        in_specs=[
            pl.BlockSpec((bb, c, hw), lambda i: (i, 0, 0)),
            pl.BlockSpec((nc, c), lambda i: (0, 0)),
        ],
        out_specs=[
            pl.BlockSpec((bb, c, hw), lambda i: (i, 0, 0)),
            pl.BlockSpec((bb, 1, hw), lambda i: (i, 0, 0)),
        ],
        out_shape=[
            jax.ShapeDtypeStruct((b, c, hw), jnp.float32),
            jax.ShapeDtypeStruct((b, 1, hw), jnp.int32),
        ],
        interpret=interpret,
    )(z3, codebook)
    return zq3.reshape(b, c, h, w), idx3.reshape(b, h, w)


# speedup vs baseline: 1.3242x; 1.3048x over previous
import functools

import jax
import jax.numpy as jnp
from jax.experimental import pallas as pl

_DN = (((1,), (0,)), ((), ()))
_IMAX = 2147483647


def _split2(x):
    x1 = x.astype(jnp.bfloat16)
    x2 = (x - x1.astype(jnp.float32)).astype(jnp.bfloat16)
    return x1, x2


def _ref_sum64(sq):
    w = sq.reshape(8, 8, sq.shape[-1])
    a = w[:, 0:4, :] + w[:, 4:8, :]
    a = a[:, 0:2, :] + a[:, 2:4, :]
    a = a[:, 0:1, :] + a[:, 1:2, :]
    s = a[0]
    for g in range(1, 8):
        s = s + a[g]
    return s


def _vq_body(z, la_cat, ct):
    t = z.shape[-1]
    nch = z.shape[0]
    ones = jnp.ones((1, t), jnp.float32)
    rhs = jnp.concatenate([z, ones], axis=0)
    r1, r2 = _split2(rhs)
    rcat = jnp.concatenate([r1, r2, r1], axis=0)
    d = jax.lax.dot_general(la_cat, rcat, _DN,
                            preferred_element_type=jnp.float32)
    iota = jax.lax.broadcasted_iota(jnp.int32, d.shape, 0)
    bits = jax.lax.bitcast_convert_type(d, jnp.int32)
    kk = jax.lax.bitcast_convert_type((bits & -512) | iota, jnp.float32)

    def top(kc):
        kmin = jnp.min(kc, axis=0, keepdims=True)
        i = jax.lax.bitcast_convert_type(kmin, jnp.int32) & 511
        return i, jnp.where(kc == kmin, jnp.inf, kc)

    i1, kk = top(kk)
    i2, kk = top(kk)
    i3, _ = top(kk)

    iota512 = jax.lax.broadcasted_iota(jnp.int32, (ct.shape[1], t), 0)

    def gather(i):
        oh = (iota512 == i).astype(jnp.bfloat16)
        p = jax.lax.dot_general(ct, oh, _DN, preferred_element_type=jnp.float32)
        return (p[0:64] + p[64:128]) + p[128:192]

    g1, g2, g3 = gather(i1), gather(i2), gather(i3)
    e1, e2, e3 = z - g1, z - g2, z - g3
    d1 = _ref_sum64(e1 * e1)
    d2 = _ref_sum64(e2 * e2)
    d3 = _ref_sum64(e3 * e3)

    dcat = jnp.concatenate([d1, d2, d3], axis=0)
    icat = jnp.concatenate([i1, i2, i3], axis=0)
    dmin = jnp.min(dcat, axis=0, keepdims=True)
    idx = jnp.min(jnp.where(dcat == dmin, icat, 512), axis=0, keepdims=True)
    e1b = jnp.broadcast_to(idx == i1, (nch, t))
    e2b = jnp.broadcast_to(idx == i2, (nch, t))
    g = jnp.where(e1b, g1, jnp.where(e2b, g2, g3))
    return z + (g - z), idx


def _vq_tc_kernel(z_ref, cb_ref, zq_ref, idx_ref):
    cb = cb_ref[...]
    cn = jnp.sum(cb * cb, axis=1, keepdims=True)
    laug = jnp.concatenate([cb * (-2.0), cn], axis=1)
    la1, la2 = _split2(laug)
    la_cat = jnp.concatenate([la1, la1, la2], axis=1)
    cbt = cb.T
    c1 = cbt.astype(jnp.bfloat16)
    r = cbt - c1.astype(jnp.float32)
    c2 = r.astype(jnp.bfloat16)
    c3 = (r - c2.astype(jnp.float32)).astype(jnp.bfloat16)
    ct = jnp.concatenate([c1, c2, c3], axis=0)
    for j in range(z_ref.shape[0]):
        zq, idx = _vq_body(z_ref[j], la_cat, ct)
        zq_ref[j] = zq
        idx_ref[j] = idx


@functools.partial(jax.jit, static_argnames=("interpret",))
def kernel(z_e, codebook, interpret=False):
    b, c, h, w = z_e.shape
    hw = h * w
    nc = codebook.shape[0]
    bb = 8
    z3 = z_e.reshape(b, c, hw)
    zq3, idx3 = pl.pallas_call(
        _vq_tc_kernel,
        grid=(b // bb,),
        in_specs=[
            pl.BlockSpec((bb, c, hw), lambda i: (i, 0, 0)),
            pl.BlockSpec((nc, c), lambda i: (0, 0)),
        ],
        out_specs=[
            pl.BlockSpec((bb, c, hw), lambda i: (i, 0, 0)),
            pl.BlockSpec((bb, 1, hw), lambda i: (i, 0, 0)),
        ],
        out_shape=[
            jax.ShapeDtypeStruct((b, c, hw), jnp.float32),
            jax.ShapeDtypeStruct((b, 1, hw), jnp.int32),
        ],
        interpret=interpret,
    )(z3, codebook)
    return zq3.reshape(b, c, h, w), idx3.reshape(b, h, w)
